# baseline (speedup 1.0000x reference)
import jax
import jax.numpy as jnp
from jax import lax
from jax.experimental import pallas as pl
from jax.experimental.pallas import tpu as pltpu
from jax.experimental.pallas import tpu_sc as plsc

MAXLEN = 200
EMB = 32
BATCH = 4096
VOCAB = 2 * MAXLEN + 2

BLK = 8
NBLK = MAXLEN // BLK
PAD_BLOCK = 402
EROWS = 416
CHUNK_ROWS = 8
CHUNK_BLKS = CHUNK_ROWS * NBLK
TRASH = CHUNK_BLKS
LANES = 16


def _build_block_indices(idx_ref, pidx_ref, start_vec, len_vec, lane, r0):
    slots = []
    for r in range(CHUNK_ROWS):
        start_s = start_vec[r0 + r]
        len_s = len_vec[r0 + r]
        jb8_lo = lane * BLK
        v_lo = jnp.clip(start_s + jb8_lo, 0, PAD_BLOCK)
        idx_lo = jnp.where(jb8_lo + BLK <= len_s, v_lo, PAD_BLOCK)
        idx_ref[pl.ds(r * NBLK, LANES)] = idx_lo
        jb8_hi = (lane + 9) * BLK
        v_hi = jnp.clip(start_s + jb8_hi, 0, PAD_BLOCK)
        idx_hi = jnp.where(jb8_hi + BLK <= len_s, v_hi, PAD_BLOCK)
        idx_ref[pl.ds(r * NBLK + 9, LANES)] = idx_hi
        rem = lax.rem(len_s, BLK)
        v_m = start_s + (len_s - rem)
        pvals = jnp.where(lane < rem, jnp.clip(v_m + lane, 0, VOCAB - 1), 0)
        pidx_ref[pl.ds(r * LANES, LANES)] = pvals
        slot = jnp.where(len_s < MAXLEN, r * NBLK + lax.div(len_s, BLK), TRASH)
        slots.append(slot)
    return slots


def _fire_patches(tab_s, pidx_ref, blk_ref, slots, psem):
    copies = []
    for r in range(CHUNK_ROWS):
        copies.append(
            pltpu.async_copy(
                tab_s.at[pidx_ref.at[pl.ds(r * LANES, BLK)]],
                blk_ref.at[slots[r]],
                psem,
            )
        )
    for cp in copies:
        cp.wait()


def _fire_outs(blk_ref, out_hbm, row_base, osem):
    for r in range(CHUNK_ROWS):
        pltpu.async_copy(
            blk_ref.at[pl.ds(r * NBLK, NBLK)],
            out_hbm.at[row_base + r, :, :, pl.ds(0, EMB)],
            osem,
        )


def _drain_outs(blk_ref, out_hbm, row_base, osem):
    for r in range(CHUNK_ROWS):
        pltpu.make_async_copy(
            blk_ref.at[pl.ds(r * NBLK, NBLK)],
            out_hbm.at[row_base + r, :, :, pl.ds(0, EMB)],
            osem,
        ).wait()


def _body(pos_hbm, len_hbm, table_hbm, out_hbm, tab_s, e_s, tabv, padv,
          pos_v, len_v, idx_a, idx_b, pidx_a, pidx_b, blk_a, blk_b,
          bsem, gsem_a, gsem_b, psem, osem_a, osem_b):
    info = plsc.get_sparse_core_info()
    nc = info.num_cores
    nw = nc * info.num_subcores
    rows_per_worker = BATCH // nw
    num_pairs = rows_per_worker // (2 * CHUNK_ROWS)

    sid = lax.axis_index("s")
    wid = sid * nc + lax.axis_index("c")
    base = wid * rows_per_worker

    @pl.when(sid == 0)
    def _():
        pltpu.sync_copy(table_hbm, tab_s)

    pltpu.sync_copy(pos_hbm.at[pl.ds(base, rows_per_worker)], pos_v)
    pltpu.sync_copy(len_hbm.at[pl.ds(base, rows_per_worker)], len_v)
    plsc.subcore_barrier()
    pltpu.sync_copy(tab_s, tabv.at[pl.ds(0, VOCAB)])
    lane = lax.iota(jnp.int32, LANES)

    for t in range(BLK):
        pltpu.sync_copy(tab_s.at[pl.ds(0, 1)], padv.at[pl.ds(t, 1)])

    bcps = []
    for i in range(EROWS // LANES):
        v = sid * 26 + i
        bcps.append(pltpu.async_copy(
            tabv.at[pl.ds(v, BLK)], e_s.at[v], bsem))
    for cp in bcps:
        cp.wait()

    @pl.when(sid == 15)
    def _():
        pltpu.sync_copy(padv, e_s.at[PAD_BLOCK])

    plsc.subcore_barrier()

    def pair_body(k, carry):
        pos_vec = pos_v[pl.ds(k * 2 * CHUNK_ROWS, LANES)]
        len_vec = len_v[pl.ds(k * 2 * CHUNK_ROWS, LANES)]
        start_vec = (MAXLEN + 1) - pos_vec
        row_a = base + k * 2 * CHUNK_ROWS
        row_b = row_a + CHUNK_ROWS

        slots_a = _build_block_indices(idx_a, pidx_a, start_vec, len_vec,
                                       lane, 0)

        @pl.when(k > 0)
        def _():
            _drain_outs(blk_a, out_hbm, row_a, osem_a)

        ga = pltpu.async_copy(e_s.at[idx_a], blk_a.at[pl.ds(0, CHUNK_BLKS)],
                              gsem_a)

        slots_b = _build_block_indices(idx_b, pidx_b, start_vec, len_vec,
                                       lane, CHUNK_ROWS)

        @pl.when(k > 0)
        def _():
            _drain_outs(blk_b, out_hbm, row_b, osem_b)

        gb = pltpu.async_copy(e_s.at[idx_b], blk_b.at[pl.ds(0, CHUNK_BLKS)],
                              gsem_b)

        ga.wait()
        _fire_patches(tab_s, pidx_a, blk_a, slots_a, psem)
        _fire_outs(blk_a, out_hbm, row_a, osem_a)
        gb.wait()
        _fire_patches(tab_s, pidx_b, blk_b, slots_b, psem)
        _fire_outs(blk_b, out_hbm, row_b, osem_b)
        return carry

    lax.fori_loop(0, num_pairs, pair_body, 0)
    last_a = base + rows_per_worker - 2 * CHUNK_ROWS
    _drain_outs(blk_a, out_hbm, last_a, osem_a)
    _drain_outs(blk_b, out_hbm, last_a + CHUNK_ROWS, osem_b)


def kernel(positions, lengths, table):
    info = plsc.get_sparse_core_info()
    nw = info.num_cores * info.num_subcores
    rows_per_worker = BATCH // nw
    mesh = plsc.VectorSubcoreMesh(core_axis_name="c", subcore_axis_name="s")
    k = pl.kernel(
        _body,
        out_type=jax.ShapeDtypeStruct((BATCH, NBLK, BLK, 128), jnp.float32),
        mesh=mesh,
        compiler_params=pltpu.CompilerParams(use_tc_tiling_on_sc=False),
        scratch_types=[
            pltpu.VMEM_SHARED((VOCAB, EMB), jnp.float32),
            pltpu.VMEM_SHARED((EROWS, BLK, EMB), jnp.float32),
            pltpu.VMEM((EROWS + BLK, EMB), jnp.float32),
            pltpu.VMEM((BLK, EMB), jnp.float32),
            pltpu.VMEM((rows_per_worker,), jnp.int32),
            pltpu.VMEM((rows_per_worker,), jnp.int32),
            pltpu.VMEM((CHUNK_BLKS,), jnp.int32),
            pltpu.VMEM((CHUNK_BLKS,), jnp.int32),
            pltpu.VMEM((CHUNK_ROWS * LANES,), jnp.int32),
            pltpu.VMEM((CHUNK_ROWS * LANES,), jnp.int32),
            pltpu.VMEM((CHUNK_BLKS + 1, BLK, EMB), jnp.float32),
            pltpu.VMEM((CHUNK_BLKS + 1, BLK, EMB), jnp.float32),
            pltpu.SemaphoreType.DMA,
            pltpu.SemaphoreType.DMA,
            pltpu.SemaphoreType.DMA,
            pltpu.SemaphoreType.DMA,
            pltpu.SemaphoreType.DMA,
            pltpu.SemaphoreType.DMA,
        ],
    )
    padded = k(positions.astype(jnp.int32), lengths.astype(jnp.int32), table)
    return padded[:, :, :, :EMB].reshape(BATCH, MAXLEN, EMB)


# speedup vs baseline: 2.1414x; 2.1414x over previous
import jax
import jax.numpy as jnp
from jax import lax
from jax.experimental import pallas as pl
from jax.experimental.pallas import tpu as pltpu
from jax.experimental.pallas import tpu_sc as plsc

MAXLEN = 200
EMB = 32
BATCH = 4096
VOCAB = 2 * MAXLEN + 2

BLK = 8
NBLK = MAXLEN // BLK
PAD_BLOCK = 402
EROWS = 416
CHUNK_ROWS = 8
CHUNK_BLKS = CHUNK_ROWS * NBLK
TRASH = CHUNK_BLKS
LANES = 16


def _build_block_indices(idx_ref, pidx_ref, start_vec, len_vec, lane, r0):
    slots = []
    for r in range(CHUNK_ROWS):
        start_s = start_vec[r0 + r]
        len_s = len_vec[r0 + r]
        jb8_lo = lane * BLK
        v_lo = jnp.clip(start_s + jb8_lo, 0, PAD_BLOCK)
        idx_lo = jnp.where(jb8_lo + BLK <= len_s, v_lo, PAD_BLOCK)
        idx_ref[pl.ds(r * NBLK, LANES)] = idx_lo
        jb8_hi = (lane + 9) * BLK
        v_hi = jnp.clip(start_s + jb8_hi, 0, PAD_BLOCK)
        idx_hi = jnp.where(jb8_hi + BLK <= len_s, v_hi, PAD_BLOCK)
        idx_ref[pl.ds(r * NBLK + 9, LANES)] = idx_hi
        rem = lax.rem(len_s, BLK)
        v_m = start_s + (len_s - rem)
        pvals = jnp.where(lane < rem, jnp.clip(v_m + lane, 0, VOCAB - 1), 0)
        pidx_ref[pl.ds(r * LANES, LANES)] = pvals
        slot = jnp.where(len_s < MAXLEN, r * NBLK + lax.div(len_s, BLK), TRASH)
        slots.append(slot)
    return slots


def _fire_patches(tab_s, pidx_ref, blk_ref, slots, psem):
    copies = []
    for r in range(CHUNK_ROWS):
        copies.append(
            pltpu.async_copy(
                tab_s.at[pidx_ref.at[pl.ds(r * LANES, BLK)]],
                blk_ref.at[slots[r]],
                psem,
            )
        )
    for cp in copies:
        cp.wait()


def _fire_outs(blk_ref, out_hbm, row_base, osem):
    for r in range(CHUNK_ROWS):
        pltpu.async_copy(
            blk_ref.at[pl.ds(r * NBLK, NBLK)],
            out_hbm.at[row_base + r, :, :, pl.ds(0, EMB)],
            osem,
        )


def _drain_outs(blk_ref, out_hbm, row_base, osem):
    for r in range(CHUNK_ROWS):
        pltpu.make_async_copy(
            blk_ref.at[pl.ds(r * NBLK, NBLK)],
            out_hbm.at[row_base + r, :, :, pl.ds(0, EMB)],
            osem,
        ).wait()


def _body(pos_hbm, len_hbm, table_hbm, out_hbm, tab_s, e_s, tabv, padv,
          pos_v, len_v, idx_a, idx_b, pidx_a, pidx_b, blk_a, blk_b,
          bsem, gsem_a, gsem_b, psem, osem_a, osem_b):
    info = plsc.get_sparse_core_info()
    nc = info.num_cores
    nw = nc * info.num_subcores
    rows_per_worker = BATCH // nw
    num_pairs = rows_per_worker // (2 * CHUNK_ROWS)

    sid = lax.axis_index("s")
    wid = sid * nc + lax.axis_index("c")
    base = wid * rows_per_worker

    @pl.when(sid == 0)
    def _():
        pltpu.sync_copy(table_hbm, tab_s)

    pltpu.sync_copy(pos_hbm.at[pl.ds(base, rows_per_worker)], pos_v)
    pltpu.sync_copy(len_hbm.at[pl.ds(base, rows_per_worker)], len_v)
    plsc.subcore_barrier()
    pltpu.sync_copy(tab_s, tabv.at[pl.ds(0, VOCAB)])
    lane = lax.iota(jnp.int32, LANES)

    for t in range(BLK):
        pltpu.sync_copy(tab_s.at[pl.ds(0, 1)], padv.at[pl.ds(t, 1)])

    bcps = []
    for i in range(EROWS // LANES):
        v = sid * 26 + i
        bcps.append(pltpu.async_copy(
            tabv.at[pl.ds(v, BLK)], e_s.at[v], bsem))
    for cp in bcps:
        cp.wait()

    @pl.when(sid == 15)
    def _():
        pltpu.sync_copy(padv, e_s.at[PAD_BLOCK])

    plsc.subcore_barrier()

    def pair_body(k, carry):
        pos_vec = pos_v[pl.ds(k * 2 * CHUNK_ROWS, LANES)]
        len_vec = len_v[pl.ds(k * 2 * CHUNK_ROWS, LANES)]
        start_vec = (MAXLEN + 1) - pos_vec
        row_a = base + k * 2 * CHUNK_ROWS
        row_b = row_a + CHUNK_ROWS

        slots_a = _build_block_indices(idx_a, pidx_a, start_vec, len_vec,
                                       lane, 0)

        @pl.when(k > 0)
        def _():
            _drain_outs(blk_a, out_hbm, row_a, osem_a)

        ga = pltpu.async_copy(e_s.at[idx_a], blk_a.at[pl.ds(0, CHUNK_BLKS)],
                              gsem_a)

        slots_b = _build_block_indices(idx_b, pidx_b, start_vec, len_vec,
                                       lane, CHUNK_ROWS)

        @pl.when(k > 0)
        def _():
            _drain_outs(blk_b, out_hbm, row_b, osem_b)

        gb = pltpu.async_copy(e_s.at[idx_b], blk_b.at[pl.ds(0, CHUNK_BLKS)],
                              gsem_b)

        ga.wait()
        _fire_patches(tab_s, pidx_a, blk_a, slots_a, psem)
        _fire_outs(blk_a, out_hbm, row_a, osem_a)
        gb.wait()
        _fire_patches(tab_s, pidx_b, blk_b, slots_b, psem)
        _fire_outs(blk_b, out_hbm, row_b, osem_b)
        return carry

    lax.fori_loop(0, num_pairs, pair_body, 0)
    last_a = base + rows_per_worker - 2 * CHUNK_ROWS
    _drain_outs(blk_a, out_hbm, last_a, osem_a)
    _drain_outs(blk_b, out_hbm, last_a + CHUNK_ROWS, osem_b)


def kernel(positions, lengths, table):
    info = plsc.get_sparse_core_info()
    nw = info.num_cores * info.num_subcores
    rows_per_worker = BATCH // nw
    mesh = plsc.VectorSubcoreMesh(core_axis_name="c", subcore_axis_name="s")
    k = pl.kernel(
        _body,
        out_type=jax.ShapeDtypeStruct((BATCH, NBLK, BLK, 128), jnp.float32),
        mesh=mesh,
        compiler_params=pltpu.CompilerParams(use_tc_tiling_on_sc=False),
        scratch_types=[
            pltpu.VMEM_SHARED((VOCAB, EMB), jnp.float32),
            pltpu.VMEM_SHARED((EROWS, BLK, EMB), jnp.float32),
            pltpu.VMEM((EROWS + BLK, EMB), jnp.float32),
            pltpu.VMEM((BLK, EMB), jnp.float32),
            pltpu.VMEM((rows_per_worker,), jnp.int32),
            pltpu.VMEM((rows_per_worker,), jnp.int32),
            pltpu.VMEM((CHUNK_BLKS,), jnp.int32),
            pltpu.VMEM((CHUNK_BLKS,), jnp.int32),
            pltpu.VMEM((CHUNK_ROWS * LANES,), jnp.int32),
            pltpu.VMEM((CHUNK_ROWS * LANES,), jnp.int32),
            pltpu.VMEM((CHUNK_BLKS + 1, BLK, EMB), jnp.float32),
            pltpu.VMEM((CHUNK_BLKS + 1, BLK, EMB), jnp.float32),
            pltpu.SemaphoreType.DMA,
            pltpu.SemaphoreType.DMA,
            pltpu.SemaphoreType.DMA,
            pltpu.SemaphoreType.DMA,
            pltpu.SemaphoreType.DMA,
            pltpu.SemaphoreType.DMA,
        ],
    )
    padded = k(positions.astype(jnp.int32), lengths.astype(jnp.int32), table)
    return padded.reshape(BATCH, MAXLEN, 128)[:, :, :EMB]
